# baseline (speedup 1.0000x reference)
import functools

import jax
import jax.numpy as jnp
from jax import lax
from jax.experimental import pallas as pl
from jax.experimental.pallas import tpu as pltpu
from jax.experimental.pallas import tpu_sc as plsc

_NC = 2
_NS = 16
_NW = _NC * _NS


_NBUF = 4
_LOOK = 2


def _sc_body(pairs_per_w, hh, a, b, o1, o2, buf, sem_in, sem_out):
    wid = lax.axis_index("s") * _NC + lax.axis_index("c")
    base = wid * pairs_per_w
    jobs = []
    for j in range(pairs_per_w):
        r = base + j
        for h0 in (0, hh):
            jobs.append((b, o1, r, 0, h0))
            jobs.append((a, o1, r, 1, h0))
            jobs.append((a, o2, r, 0, h0))
            jobs.append((b, o2, r, 1, h0))
    nj = len(jobs)

    def start_in(i, slot):
        src, _, r, s, h0 = jobs[i]
        return pltpu.async_copy(src.at[r, s, pl.ds(h0, hh)],
                                buf.at[slot], sem_in)

    def start_out(i, slot):
        _, dst, r, s, h0 = jobs[i]
        return pltpu.async_copy(buf.at[slot],
                                dst.at[r, s, pl.ds(h0, hh)], sem_out)

    ins = [None] * _NBUF
    outs = [None] * _NBUF
    for i in range(min(_LOOK, nj)):
        ins[i % _NBUF] = start_in(i, i % _NBUF)
    for i in range(nj):
        s = i % _NBUF
        ip = i + _LOOK
        if ip < nj:
            ps = ip % _NBUF
            if outs[ps] is not None:
                outs[ps].wait()
                outs[ps] = None
            ins[ps] = start_in(ip, ps)
        ins[s].wait()
        outs[s] = start_out(i, s)
    for o in outs:
        if o is not None:
            o.wait()


def kernel(x1, x2):
    N, C, H, W = x1.shape
    R = N * C // 2
    pairs_per_w = R // _NW
    a = x1.reshape(R, 2, H, W)
    b = x2.reshape(R, 2, H, W)
    hh = H // 2
    mesh = plsc.VectorSubcoreMesh(core_axis_name="c", subcore_axis_name="s")
    run = pl.kernel(
        functools.partial(_sc_body, pairs_per_w, hh),
        out_type=[jax.ShapeDtypeStruct((R, 2, H, W), jnp.float32)] * 2,
        mesh=mesh,
        scratch_types=[
            pltpu.VMEM((_NBUF, H // 2, W), jnp.float32),
            pltpu.SemaphoreType.DMA,
            pltpu.SemaphoreType.DMA,
        ],
    )
    o1, o2 = run(a, b)
    return o1.reshape(N, C, H, W), o2.reshape(N, C, H, W)


# speedup vs baseline: 1.1026x; 1.1026x over previous
import functools

import jax
import jax.numpy as jnp
from jax import lax
from jax.experimental import pallas as pl
from jax.experimental.pallas import tpu as pltpu
from jax.experimental.pallas import tpu_sc as plsc

_NC = 2
_NS = 16
_NW = _NC * _NS
_NBUF = 4
_LOOK = 2


def _sc_body(pairs_per_w, hh, a, b, o1, buf, sem_in, sem_out):
    wid = lax.axis_index("s") * _NC + lax.axis_index("c")
    base = wid * pairs_per_w
    jobs = []
    for j in range(pairs_per_w):
        r = base + j
        for h0 in (0, hh):
            jobs.append((b, r, 0, h0))
            jobs.append((a, r, 1, h0))
    nj = len(jobs)

    def start_in(i, slot):
        src, r, s, h0 = jobs[i]
        return pltpu.async_copy(src.at[r, s, pl.ds(h0, hh)],
                                buf.at[slot], sem_in)

    def start_out(i, slot):
        _, r, s, h0 = jobs[i]
        return pltpu.async_copy(buf.at[slot],
                                o1.at[r, s, pl.ds(h0, hh)], sem_out)

    ins = [None] * _NBUF
    outs = [None] * _NBUF
    for i in range(min(_LOOK, nj)):
        ins[i % _NBUF] = start_in(i, i % _NBUF)
    for i in range(nj):
        s = i % _NBUF
        ip = i + _LOOK
        if ip < nj:
            ps = ip % _NBUF
            if outs[ps] is not None:
                outs[ps].wait()
                outs[ps] = None
            ins[ps] = start_in(ip, ps)
        ins[s].wait()
        outs[s] = start_out(i, s)
    for o in outs:
        if o is not None:
            o.wait()


def _tc_body(a_ref, b_ref, o2_ref):
    o2_ref[:, 0] = a_ref[:, 0]
    o2_ref[:, 1] = b_ref[:, 0]


def kernel(x1, x2):
    N, C, H, W = x1.shape
    R = N * C // 2
    pairs_per_w = R // _NW
    hh = H // 2
    a = x1.reshape(R, 2, H, W)
    b = x2.reshape(R, 2, H, W)

    mesh = plsc.VectorSubcoreMesh(core_axis_name="c", subcore_axis_name="s")
    sc_run = pl.kernel(
        functools.partial(_sc_body, pairs_per_w, hh),
        out_type=jax.ShapeDtypeStruct((R, 2, H, W), jnp.float32),
        mesh=mesh,
        scratch_types=[
            pltpu.VMEM((_NBUF, hh, W), jnp.float32),
            pltpu.SemaphoreType.DMA,
            pltpu.SemaphoreType.DMA,
        ],
    )
    o1 = sc_run(a, b)

    BP = 4
    o2 = pl.pallas_call(
        _tc_body,
        grid=(R // BP,),
        in_specs=[
            pl.BlockSpec((BP, 1, H, W), lambda i: (i, 0, 0, 0)),
            pl.BlockSpec((BP, 1, H, W), lambda i: (i, 1, 0, 0)),
        ],
        out_specs=pl.BlockSpec((BP, 2, H, W), lambda i: (i, 0, 0, 0)),
        out_shape=jax.ShapeDtypeStruct((R, 2, H, W), jnp.float32),
    )(a, b)

    return o1.reshape(N, C, H, W), o2.reshape(N, C, H, W)
